# baseline (speedup 1.0000x reference)
import functools

import jax
import jax.numpy as jnp
from jax import lax
from jax.experimental import pallas as pl
from jax.experimental.pallas import tpu as pltpu
from jax.experimental.pallas import tpu_sc as plsc


def _router_kernel(x_ref, rw_ref, rb_ref, xaug_ref, pos_ref, cnt_ref,
                   aux_ref, *, T, E, capacity, alpha, q_chunk):
    x = x_ref[...]
    logits = jax.lax.dot_general(
        x, rw_ref[...], (((1,), (1,)), ((), ())),
        preferred_element_type=jnp.float32) + rb_ref[0:1, :]
    m = jnp.max(logits, axis=1, keepdims=True)
    ex = jnp.exp(logits - m)
    probs = ex / jnp.sum(ex, axis=1, keepdims=True)

    p = jnp.max(probs, axis=1, keepdims=True)
    e_iota = jax.lax.broadcasted_iota(jnp.int32, (T, E), 1)
    eidx = jnp.min(jnp.where(probs == p, e_iota, E), axis=1,
                   keepdims=True)

    one_hot_p = jnp.where(e_iota == eidx, p, 0.0)
    f_sum = jnp.sum(one_hot_p, axis=0, keepdims=True)
    p_sum = jnp.sum(probs, axis=0, keepdims=True)
    aux_ref[...] = (alpha * E / (T * T)) * jnp.sum(f_sum * p_sum,
                                                   keepdims=True)

    oh128 = jax.lax.broadcasted_iota(jnp.int32, (T, 128), 1) == eidx
    cnt_f = jnp.sum(jnp.where(oh128, 1.0, 0.0), axis=0, keepdims=True)
    cnt_ref[...] = cnt_f.astype(jnp.int32)

    p_row = jnp.transpose(p)
    e_row = jnp.transpose(eidx)
    k_idx = jax.lax.broadcasted_iota(jnp.int32, (1, T), 1)
    for c0 in range(0, T, q_chunk):
        pq = p[c0:c0 + q_chunk]
        eq = eidx[c0:c0 + q_chunk]
        qi = jax.lax.broadcasted_iota(jnp.int32, (q_chunk, 1), 0) + c0
        ematch = e_row == eq
        before = (p_row > pq) | ((p_row == pq) & (k_idx <= qi))
        mass = jnp.where(before & ematch, p_row, 0.0)
        s = jnp.sum(mass, axis=1, keepdims=True)
        keep = (s <= capacity).astype(jnp.float32)
        scale = keep * pq
        posmask = (e_row < eq) | (ematch & (k_idx < qi))
        posr = jnp.sum(jnp.where(posmask, 1.0, 0.0),
                       axis=1, keepdims=True)
        pos_ref[c0:c0 + q_chunk, :] = posr.astype(jnp.int32)
        xq = x[c0:c0 + q_chunk]
        xaug_ref[c0:c0 + q_chunk, 0:x.shape[1]] = scale * xq
        xaug_ref[c0:c0 + q_chunk, x.shape[1]:] = jnp.where(
            oh128[c0:c0 + q_chunk], scale, 0.0)


def _make_sc_scatter(T, KA, NW, CH):
    mesh = plsc.VectorSubcoreMesh(core_axis_name="c", subcore_axis_name="s")

    @functools.partial(
        pl.kernel, mesh=mesh,
        out_type=jax.ShapeDtypeStruct((T, KA), jnp.float32),
        scratch_types=[
            pltpu.VMEM((CH,), jnp.int32),
            pltpu.VMEM((CH, KA), jnp.float32),
            pltpu.SemaphoreType.DMA,
        ],
    )
    def sc_scatter(xaug_hbm, pos_hbm, xs_hbm, pos_v, rows_v, sem):
        nc = 2
        wid = lax.axis_index("s") * nc + lax.axis_index("c")
        base = wid * CH
        pltpu.sync_copy(pos_hbm.at[pl.ds(base, CH)], pos_v)
        pltpu.sync_copy(xaug_hbm.at[pl.ds(base, CH)], rows_v)
        pltpu.async_copy(rows_v, xs_hbm.at[pos_v], sem).wait()

    return sc_scatter


def _make_sc_gather(T, D, NW, CH):
    mesh = plsc.VectorSubcoreMesh(core_axis_name="c", subcore_axis_name="s")

    @functools.partial(
        pl.kernel, mesh=mesh,
        out_type=jax.ShapeDtypeStruct((T, D), jnp.float32),
        scratch_types=[
            pltpu.VMEM((CH,), jnp.int32),
            pltpu.VMEM((CH, D), jnp.float32),
            pltpu.SemaphoreType.DMA,
        ],
    )
    def sc_gather(ys_hbm, pos_hbm, out_hbm, pos_v, rows_v, sem):
        nc = 2
        wid = lax.axis_index("s") * nc + lax.axis_index("c")
        base = wid * CH
        pltpu.sync_copy(pos_hbm.at[pl.ds(base, CH)], pos_v)
        pltpu.async_copy(ys_hbm.at[pos_v], rows_v, sem).wait()
        pltpu.sync_copy(rows_v, out_hbm.at[pl.ds(base, CH)])

    return sc_gather


def _grouped_kernel(tile_r, eid_r, valid_r, rlo_r, rhi_r, first_r,
                    xs_ref, ew_ref, ebt_ref, ys_ref, *, TB, D):
    s = pl.program_id(0)
    xs = xs_ref[...]
    w = ew_ref[0]
    y = jax.lax.dot_general(xs[:, 0:D], w, (((1,), (1,)), ((), ())),
                            preferred_element_type=jnp.float32)
    bcols = jnp.where(
        jax.lax.broadcasted_iota(jnp.int32, (D, 128), 1) == eid_r[s],
        ebt_ref[0], 0.0)
    y = y + jax.lax.dot_general(xs[:, D:], bcols, (((1,), (1,)), ((), ())),
                                preferred_element_type=jnp.float32)
    r = jax.lax.broadcasted_iota(jnp.int32, (TB, 1), 0) + tile_r[s] * TB
    ok = (valid_r[s] == 1) & (r >= rlo_r[s]) & (r < rhi_r[s])
    contrib = jnp.where(ok, y, 0.0)

    @pl.when(first_r[s] == 1)
    def _():
        ys_ref[...] = contrib

    @pl.when(first_r[s] != 1)
    def _():
        ys_ref[...] += contrib


def kernel(x, router_w, router_b, expert_w, expert_b):
    B, S, D = x.shape
    E = router_w.shape[0]
    T = B * S
    capacity = float(int(T / E * 1.0))
    alpha = 0.01
    KA = D + 128
    NW, CH = 32, T // 32
    TB = 128
    NT = T // TB
    NP = NT + E - 1

    xf = x.reshape(T, D)
    rb2 = router_b.reshape(1, E)

    xaug, pos2, cnt, aux = pl.pallas_call(
        functools.partial(_router_kernel, T=T, E=E, capacity=capacity,
                          alpha=alpha, q_chunk=256),
        out_shape=[
            jax.ShapeDtypeStruct((T, KA), jnp.float32),
            jax.ShapeDtypeStruct((T, 1), jnp.int32),
            jax.ShapeDtypeStruct((1, 128), jnp.int32),
            jax.ShapeDtypeStruct((1, 1), jnp.float32),
        ],
    )(xf, router_w, rb2)

    counts = cnt[0, :E]
    off = jnp.concatenate([jnp.zeros((1,), jnp.int32), jnp.cumsum(counts)])
    tile_lo = off[:E] // TB
    tile_hi = jnp.maximum(off[1:] - 1, 0) // TB
    ntiles = jnp.where(counts > 0, tile_hi - tile_lo + 1, 0)
    pstart = jnp.concatenate([jnp.zeros((1,), jnp.int32),
                              jnp.cumsum(ntiles)])
    ptot = pstart[E]
    sidx = jnp.arange(NP, dtype=jnp.int32)
    e_s = jnp.sum((sidx[:, None] >= pstart[None, :E]).astype(jnp.int32),
                  axis=1) - 1
    valid = (sidx < ptot).astype(jnp.int32)
    e_s = jnp.where(valid == 1, e_s, E - 1)
    tile_s = jnp.where(valid == 1, tile_lo[e_s] + (sidx - pstart[e_s]),
                       NT - 1)
    rlo = off[e_s]
    rhi = off[e_s + 1]
    prev = jnp.concatenate([jnp.full((1,), -1, jnp.int32), tile_s[:-1]])
    first = ((tile_s != prev) & (valid == 1)).astype(jnp.int32)

    pos = pos2.reshape(T)
    xs = _make_sc_scatter(T, KA, NW, CH)(xaug, pos)

    grid_spec = pltpu.PrefetchScalarGridSpec(
        num_scalar_prefetch=6,
        grid=(NP,),
        in_specs=[
            pl.BlockSpec((TB, KA), lambda s, t, e, v, l, h, f: (t[s], 0)),
            pl.BlockSpec((1, D, D), lambda s, t, e, v, l, h, f: (e[s], 0, 0)),
            pl.BlockSpec((1, D, 1), lambda s, t, e, v, l, h, f: (e[s], 0, 0)),
        ],
        out_specs=pl.BlockSpec((TB, D), lambda s, t, e, v, l, h, f: (t[s], 0)),
    )
    ys = pl.pallas_call(
        functools.partial(_grouped_kernel, TB=TB, D=D),
        grid_spec=grid_spec,
        out_shape=jax.ShapeDtypeStruct((T, D), jnp.float32),
    )(tile_s, e_s, valid, rlo, rhi, first,
      xs, expert_w, expert_b.reshape(E, D, 1))

    out = _make_sc_gather(T, D, NW, CH)(ys, pos)
    return out.reshape(B, S, D), aux[0, 0]


# speedup vs baseline: 1.5968x; 1.5968x over previous
import functools

import jax
import jax.numpy as jnp
from jax.experimental import pallas as pl
from jax.experimental.pallas import tpu as pltpu


def _fused_kernel(x_ref, rw_ref, rb_ref, ew_ref, eb_ref, out_ref, aux_ref,
                  scale_ref, eidx_ref, *, T, E, TBM, capacity, alpha, q_chunk):
    mi = pl.program_id(0)
    e = pl.program_id(1)
    x = x_ref[...]

    @pl.when((mi == 0) & (e == 0))
    def _router():
        logits = jax.lax.dot_general(
            x, rw_ref[...], (((1,), (1,)), ((), ())),
            preferred_element_type=jnp.float32) + rb_ref[0:1, :]
        m = jnp.max(logits, axis=1, keepdims=True)
        ex = jnp.exp(logits - m)
        probs = ex / jnp.sum(ex, axis=1, keepdims=True)

        p = jnp.max(probs, axis=1, keepdims=True)
        e_iota = jax.lax.broadcasted_iota(jnp.int32, (T, E), 1)
        eidx = jnp.min(jnp.where(probs == p, e_iota, E), axis=1,
                       keepdims=True)
        eidx_ref[...] = eidx

        one_hot_p = jnp.where(e_iota == eidx, p, 0.0)
        f_sum = jnp.sum(one_hot_p, axis=0, keepdims=True)
        p_sum = jnp.sum(probs, axis=0, keepdims=True)
        aux_ref[...] = (alpha * E / (T * T)) * jnp.sum(f_sum * p_sum,
                                                       keepdims=True)

        p_row = jnp.transpose(p)
        e_row = jnp.transpose(eidx)
        k_idx = jax.lax.broadcasted_iota(jnp.int32, (1, T), 1)
        for c0 in range(0, T, q_chunk):
            pq = p[c0:c0 + q_chunk]
            eq = eidx[c0:c0 + q_chunk]
            qi = jax.lax.broadcasted_iota(jnp.int32, (q_chunk, 1), 0) + c0
            before = (p_row > pq) | ((p_row == pq) & (k_idx <= qi))
            mass = jnp.where(before & (e_row == eq), p_row, 0.0)
            s = jnp.sum(mass, axis=1, keepdims=True)
            keep = (s <= capacity).astype(jnp.float32)
            scale_ref[c0:c0 + q_chunk, :] = keep * pq

    w = ew_ref[0]
    xt = x_ref[pl.ds(mi * TBM, TBM), :]
    y = jax.lax.dot_general(xt, w, (((1,), (1,)), ((), ())),
                            preferred_element_type=jnp.float32)
    y = y + eb_ref[0]
    m = jnp.where(eidx_ref[pl.ds(mi * TBM, TBM), :] == e,
                  scale_ref[pl.ds(mi * TBM, TBM), :], 0.0)
    contrib = m * y

    @pl.when(e == 0)
    def _():
        out_ref[...] = contrib

    @pl.when(e != 0)
    def _():
        out_ref[...] += contrib


def kernel(x, router_w, router_b, expert_w, expert_b):
    B, S, D = x.shape
    E = router_w.shape[0]
    T = B * S
    capacity = float(int(T / E * 1.0))
    alpha = 0.01

    xf = x.reshape(T, D)
    rb2 = router_b.reshape(1, E)

    MT = 2
    TBM = T // MT
    out, aux = pl.pallas_call(
        functools.partial(_fused_kernel, T=T, E=E, TBM=TBM,
                          capacity=capacity, alpha=alpha, q_chunk=256),
        grid=(MT, E),
        in_specs=[
            pl.BlockSpec((T, D), lambda m, e: (0, 0)),
            pl.BlockSpec((E, D), lambda m, e: (0, 0)),
            pl.BlockSpec((1, E), lambda m, e: (0, 0)),
            pl.BlockSpec((1, D, D), lambda m, e: (e, 0, 0)),
            pl.BlockSpec((1, 1, D), lambda m, e: (e, 0, 0)),
        ],
        out_specs=[
            pl.BlockSpec((TBM, D), lambda m, e: (m, 0)),
            pl.BlockSpec((1, 1), lambda m, e: (0, 0)),
        ],
        out_shape=[
            jax.ShapeDtypeStruct((T, D), jnp.float32),
            jax.ShapeDtypeStruct((1, 1), jnp.float32),
        ],
        scratch_shapes=[
            pltpu.VMEM((T, 1), jnp.float32),
            pltpu.VMEM((T, 1), jnp.int32),
        ],
    )(xf, router_w, rb2, expert_w, expert_b.reshape(E, 1, D))

    return out.reshape(B, S, D), aux[0, 0]


# speedup vs baseline: 1.7760x; 1.1122x over previous
import functools

import jax
import jax.numpy as jnp
from jax.experimental import pallas as pl
from jax.experimental.pallas import tpu as pltpu


def _fused_kernel(x_ref, rw_ref, rb_ref, ew_ref, eb_ref, out_ref, aux_ref,
                  scale_ref, eidx_ref, *, T, E, capacity, alpha, q_chunk):
    e = pl.program_id(0)
    x = x_ref[...]

    @pl.when(e == 0)
    def _router():
        logits = jax.lax.dot_general(
            x, rw_ref[...], (((1,), (1,)), ((), ())),
            preferred_element_type=jnp.float32) + rb_ref[0:1, :]
        m = jnp.max(logits, axis=1, keepdims=True)
        ex = jnp.exp(logits - m)
        probs = ex / jnp.sum(ex, axis=1, keepdims=True)

        p = jnp.max(probs, axis=1, keepdims=True)
        e_iota = jax.lax.broadcasted_iota(jnp.int32, (T, E), 1)
        eidx = jnp.min(jnp.where(probs == p, e_iota, E), axis=1,
                       keepdims=True)
        eidx_ref[...] = eidx

        one_hot_p = jnp.where(e_iota == eidx, p, 0.0)
        f_sum = jnp.sum(one_hot_p, axis=0, keepdims=True)
        p_sum = jnp.sum(probs, axis=0, keepdims=True)
        aux_ref[...] = (alpha * E / (T * T)) * jnp.sum(f_sum * p_sum,
                                                       keepdims=True)

        p_row = jnp.transpose(p)
        e_row = jnp.transpose(eidx)
        k_idx = jax.lax.broadcasted_iota(jnp.int32, (1, T), 1)
        for c0 in range(0, T, q_chunk):
            pq = p[c0:c0 + q_chunk]
            eq = eidx[c0:c0 + q_chunk]
            qi = jax.lax.broadcasted_iota(jnp.int32, (q_chunk, 1), 0) + c0
            before = (p_row > pq) | ((p_row == pq) & (k_idx <= qi))
            mass = jnp.where(before & (e_row == eq), p_row, 0.0)
            s = jnp.sum(mass, axis=1, keepdims=True)
            keep = (s <= capacity).astype(jnp.float32)
            scale_ref[c0:c0 + q_chunk, :] = keep * pq

    w = ew_ref[0]
    y = jax.lax.dot_general(x, w, (((1,), (1,)), ((), ())),
                            preferred_element_type=jnp.float32)
    y = y + eb_ref[0]
    m = jnp.where(eidx_ref[...] == e, scale_ref[...], 0.0)
    contrib = m * y

    @pl.when(e == 0)
    def _():
        out_ref[...] = contrib

    @pl.when(e != 0)
    def _():
        out_ref[...] += contrib


def kernel(x, router_w, router_b, expert_w, expert_b):
    B, S, D = x.shape
    E = router_w.shape[0]
    T = B * S
    capacity = float(int(T / E * 1.0))
    alpha = 0.01

    xf = x.reshape(T, D)
    rb2 = router_b.reshape(1, E)

    out, aux = pl.pallas_call(
        functools.partial(_fused_kernel, T=T, E=E, capacity=capacity,
                          alpha=alpha, q_chunk=512),
        grid=(E,),
        in_specs=[
            pl.BlockSpec((T, D), lambda e: (0, 0)),
            pl.BlockSpec((E, D), lambda e: (0, 0)),
            pl.BlockSpec((1, E), lambda e: (0, 0)),
            pl.BlockSpec((1, D, D), lambda e: (e, 0, 0)),
            pl.BlockSpec((1, 1, D), lambda e: (e, 0, 0)),
        ],
        out_specs=[
            pl.BlockSpec((T, D), lambda e: (0, 0)),
            pl.BlockSpec((1, 1), lambda e: (0, 0)),
        ],
        out_shape=[
            jax.ShapeDtypeStruct((T, D), jnp.float32),
            jax.ShapeDtypeStruct((1, 1), jnp.float32),
        ],
        scratch_shapes=[
            pltpu.VMEM((T, 1), jnp.float32),
            pltpu.VMEM((T, 1), jnp.int32),
        ],
    )(xf, router_w, rb2, expert_w, expert_b.reshape(E, 1, D))

    return out.reshape(B, S, D), aux[0, 0]


# speedup vs baseline: 1.9538x; 1.1001x over previous
import functools

import jax
import jax.numpy as jnp
from jax.experimental import pallas as pl
from jax.experimental.pallas import tpu as pltpu


def _fused_kernel(x_ref, rw_ref, rb_ref, ew_ref, eb_ref, out_ref, aux_ref,
                  scale_ref, eidx_ref, *, T, E, capacity, alpha, q_chunk):
    e = pl.program_id(0)
    x = x_ref[...]

    @pl.when(e == 0)
    def _router():
        logits = jax.lax.dot_general(
            x, rw_ref[...], (((1,), (1,)), ((), ())),
            preferred_element_type=jnp.float32) + rb_ref[0:1, :]
        m = jnp.max(logits, axis=1, keepdims=True)
        ex = jnp.exp(logits - m)
        probs = ex / jnp.sum(ex, axis=1, keepdims=True)

        p = jnp.max(probs, axis=1, keepdims=True)
        e_iota = jax.lax.broadcasted_iota(jnp.int32, (T, E), 1)
        eidx = jnp.min(jnp.where(probs == p, e_iota, E), axis=1,
                       keepdims=True)
        eidx_ref[...] = eidx

        one_hot_p = jnp.where(e_iota == eidx, p, 0.0)
        f_sum = jnp.sum(one_hot_p, axis=0, keepdims=True)
        p_sum = jnp.sum(probs, axis=0, keepdims=True)
        aux_ref[...] = (alpha * E / (T * T)) * jnp.sum(f_sum * p_sum,
                                                       keepdims=True)

        scale_ref[...] = p
        overflow = jnp.max(f_sum) > capacity

        @pl.when(overflow)
        def _capacity():
            p_row = jnp.transpose(p)
            e_row = jnp.transpose(eidx)
            k_idx = jax.lax.broadcasted_iota(jnp.int32, (1, T), 1)
            for c0 in range(0, T, q_chunk):
                pq = p[c0:c0 + q_chunk]
                eq = eidx[c0:c0 + q_chunk]
                qi = jax.lax.broadcasted_iota(
                    jnp.int32, (q_chunk, 1), 0) + c0
                before = (p_row > pq) | ((p_row == pq) & (k_idx <= qi))
                mass = jnp.where(before & (e_row == eq), p_row, 0.0)
                s = jnp.sum(mass, axis=1, keepdims=True)
                keep = (s <= capacity).astype(jnp.float32)
                scale_ref[c0:c0 + q_chunk, :] = keep * pq

    w = ew_ref[0]
    y = jax.lax.dot_general(x, w, (((1,), (1,)), ((), ())),
                            preferred_element_type=jnp.float32)
    y = y + eb_ref[0]
    m = jnp.where(eidx_ref[...] == e, scale_ref[...], 0.0)
    contrib = m * y

    @pl.when(e == 0)
    def _():
        out_ref[...] = contrib

    @pl.when(e != 0)
    def _():
        out_ref[...] += contrib


def kernel(x, router_w, router_b, expert_w, expert_b):
    B, S, D = x.shape
    E = router_w.shape[0]
    T = B * S
    capacity = float(int(T / E * 1.0))
    alpha = 0.01

    xf = x.reshape(T, D)
    rb2 = router_b.reshape(1, E)

    out, aux = pl.pallas_call(
        functools.partial(_fused_kernel, T=T, E=E, capacity=capacity,
                          alpha=alpha, q_chunk=512),
        grid=(E,),
        in_specs=[
            pl.BlockSpec((T, D), lambda e: (0, 0)),
            pl.BlockSpec((E, D), lambda e: (0, 0)),
            pl.BlockSpec((1, E), lambda e: (0, 0)),
            pl.BlockSpec((1, D, D), lambda e: (e, 0, 0)),
            pl.BlockSpec((1, 1, D), lambda e: (e, 0, 0)),
        ],
        out_specs=[
            pl.BlockSpec((T, D), lambda e: (0, 0)),
            pl.BlockSpec((1, 1), lambda e: (0, 0)),
        ],
        out_shape=[
            jax.ShapeDtypeStruct((T, D), jnp.float32),
            jax.ShapeDtypeStruct((1, 1), jnp.float32),
        ],
        scratch_shapes=[
            pltpu.VMEM((T, 1), jnp.float32),
            pltpu.VMEM((T, 1), jnp.int32),
        ],
    )(xf, router_w, rb2, expert_w, expert_b.reshape(E, 1, D))

    return out.reshape(B, S, D), aux[0, 0]
